# baseline (speedup 1.0000x reference)
import jax
import jax.numpy as jnp
from jax import lax
from jax.experimental import pallas as pl
from jax.experimental.pallas import tpu as pltpu
from jax.experimental.pallas import tpu_sc as plsc

_BATCH = 4096
_HIST = 200
_D = 64
_N = _BATCH * _HIST
_NC = 2
_NS = 16
_NW = _NC * _NS
_NPW = _N // _NW
_C = 512
_G = _NPW // _C


def _body(x_hbm, table_hbm, out_hbm, idx_v, rows_v, gsem):
    wid = lax.axis_index("s") * _NC + lax.axis_index("c")
    base = wid * _NPW
    pltpu.sync_copy(x_hbm.at[pl.ds(base, _NPW)], idx_v)

    @pl.loop(0, _G)
    def _chunk(g):
        off = g * _C
        pltpu.async_copy(
            table_hbm.at[idx_v.at[pl.ds(off, _C)]], rows_v, gsem
        ).wait()
        pltpu.sync_copy(rows_v, out_hbm.at[pl.ds(base + off, _C)])


@jax.jit
def _lookup(x_flat, table):
    mesh = plsc.VectorSubcoreMesh(core_axis_name="c", subcore_axis_name="s")
    return pl.kernel(
        _body,
        out_type=jax.ShapeDtypeStruct((_N, _D), jnp.float32),
        mesh=mesh,
        scratch_types=[
            pltpu.VMEM((_NPW,), jnp.int32),
            pltpu.VMEM((_C, _D), jnp.float32),
            pltpu.SemaphoreType.DMA,
        ],
        compiler_params=pltpu.CompilerParams(use_tc_tiling_on_sc=False),
    )(x_flat, table)


def kernel(x, table):
    out = _lookup(x.reshape(-1), table)
    return out.reshape(_BATCH, _HIST, _D)


# speedup vs baseline: 1.0231x; 1.0231x over previous
import jax
import jax.numpy as jnp
from jax import lax
from jax.experimental import pallas as pl
from jax.experimental.pallas import tpu as pltpu
from jax.experimental.pallas import tpu_sc as plsc

_BATCH = 4096
_HIST = 200
_D = 64
_N = _BATCH * _HIST
_NC = 2
_NS = 16
_NW = _NC * _NS
_NPW = _N // _NW
_C = 320
_G = _NPW // _C


_NBUF = 4


def _body(x_hbm, table_hbm, out_hbm, idx_v, rows, gsems, wsems):
    wid = lax.axis_index("s") * _NC + lax.axis_index("c")
    base = wid * _NPW
    pltpu.sync_copy(x_hbm.at[pl.ds(base, _NPW)], idx_v)

    def _gather_start(g, b):
        pltpu.async_copy(
            table_hbm.at[idx_v.at[pl.ds(g * _C, _C)]], rows[b], gsems[b]
        )

    def _write_start(g, b):
        pltpu.async_copy(
            rows[b], out_hbm.at[pl.ds(base + g * _C, _C)], wsems[b]
        )

    for b in range(_NBUF):
        _gather_start(b, b)

    @pl.loop(0, _G, step=_NBUF)
    def _outer(g0):
        for b in range(_NBUF):
            g = g0 + b
            pltpu.make_async_copy(
                table_hbm.at[idx_v.at[pl.ds(g * _C, _C)]], rows[b], gsems[b]
            ).wait()
            _write_start(g, b)
            @pl.when(g + _NBUF < _G)
            def _():
                pltpu.make_async_copy(
                    rows[b], out_hbm.at[pl.ds(base + g * _C, _C)], wsems[b]
                ).wait()
                _gather_start(g + _NBUF, b)

    for b in range(_NBUF):
        g_last = _G - _NBUF + b
        pltpu.make_async_copy(
            rows[b], out_hbm.at[pl.ds(base + g_last * _C, _C)], wsems[b]
        ).wait()


@jax.jit
def _lookup(x_flat, table):
    mesh = plsc.VectorSubcoreMesh(core_axis_name="c", subcore_axis_name="s")
    return pl.kernel(
        _body,
        out_type=jax.ShapeDtypeStruct((_N, _D), jnp.float32),
        mesh=mesh,
        scratch_types=[
            pltpu.VMEM((_NPW,), jnp.int32),
            [pltpu.VMEM((_C, _D), jnp.float32) for _ in range(_NBUF)],
            [pltpu.SemaphoreType.DMA for _ in range(_NBUF)],
            [pltpu.SemaphoreType.DMA for _ in range(_NBUF)],
        ],
        compiler_params=pltpu.CompilerParams(use_tc_tiling_on_sc=False),
    )(x_flat, table)


def kernel(x, table):
    out = _lookup(x.reshape(-1), table)
    return out.reshape(_BATCH, _HIST, _D)


# speedup vs baseline: 1.2499x; 1.2217x over previous
import jax
import jax.numpy as jnp
from jax import lax
from jax.experimental import pallas as pl
from jax.experimental.pallas import tpu as pltpu
from jax.experimental.pallas import tpu_sc as plsc

_BATCH = 4096
_HIST = 200
_D = 64
_DP = 128
_N = _BATCH * _HIST
_NC = 2
_NS = 16
_NW = _NC * _NS
_NPW = _N // _NW
_NBUF = 4
_C = 200
_G = _NPW // _C


def _body(x_hbm, table_hbm, out_hbm, idx_v, rows, gsems, wsems):
    wid = lax.axis_index("s") * _NC + lax.axis_index("c")
    base = wid * _NPW
    pltpu.sync_copy(x_hbm.at[pl.ds(base, _NPW)], idx_v)

    def _gather_start(g, b):
        pltpu.async_copy(
            table_hbm.at[idx_v.at[pl.ds(g * _C, _C)]], rows[b], gsems[b]
        )

    for b in range(_NBUF):
        _gather_start(b, b)

    @pl.loop(0, _G, step=_NBUF)
    def _outer(g0):
        for b in range(_NBUF):
            g = g0 + b
            pltpu.make_async_copy(
                table_hbm.at[idx_v.at[pl.ds(g * _C, _C)]], rows[b], gsems[b]
            ).wait()
            pltpu.async_copy(
                rows[b], out_hbm.at[pl.ds(base + g * _C, _C)], wsems[b]
            )
            @pl.when(g + _NBUF < _G)
            def _():
                pltpu.make_async_copy(
                    rows[b], out_hbm.at[pl.ds(base + g * _C, _C)], wsems[b]
                ).wait()
                _gather_start(g + _NBUF, b)

    for b in range(_NBUF):
        g_last = _G - _NBUF + b
        pltpu.make_async_copy(
            rows[b], out_hbm.at[pl.ds(base + g_last * _C, _C)], wsems[b]
        ).wait()


@jax.jit
def _lookup(x_flat, table_pad):
    mesh = plsc.VectorSubcoreMesh(core_axis_name="c", subcore_axis_name="s")
    return pl.kernel(
        _body,
        out_type=jax.ShapeDtypeStruct((_N, _DP), jnp.float32),
        mesh=mesh,
        scratch_types=[
            pltpu.VMEM((_NPW,), jnp.int32),
            [pltpu.VMEM((_C, _DP), jnp.float32) for _ in range(_NBUF)],
            [pltpu.SemaphoreType.DMA for _ in range(_NBUF)],
            [pltpu.SemaphoreType.DMA for _ in range(_NBUF)],
        ],
        compiler_params=pltpu.CompilerParams(use_tc_tiling_on_sc=True),
    )(x_flat, table_pad)


def kernel(x, table):
    table_pad = jnp.pad(table, ((0, 0), (0, _DP - _D)))
    out = _lookup(x.reshape(-1), table_pad)
    return out.reshape(_BATCH, _HIST, _DP)[:, :, :_D]
